# baseline (speedup 1.0000x reference)
import functools

import jax
import jax.numpy as jnp
from jax.experimental import pallas as pl
from jax.experimental.pallas import tpu as pltpu

NUM_EXPERTS = 8
TOP_K = 2
INPUT_SIZE = 768
HIDDEN_SIZE = 512
T_CHUNK = 512


def _router_kernel(x_ref, wg_ref, gates_ref, loss_ref):
    x = x_ref[...]
    logits = jnp.dot(x, wg_ref[...], preferred_element_type=jnp.float32)
    T = logits.shape[0]
    col = jax.lax.broadcasted_iota(jnp.int32, logits.shape, 1)

    m1 = jnp.max(logits, axis=-1, keepdims=True)
    eq1 = logits == m1
    i1 = jnp.min(jnp.where(eq1, col, NUM_EXPERTS), axis=-1, keepdims=True)
    mask1 = col == i1

    neg = jnp.full_like(logits, -jnp.inf)
    masked = jnp.where(mask1, neg, logits)
    m2 = jnp.max(masked, axis=-1, keepdims=True)
    eq2 = masked == m2
    i2 = jnp.min(jnp.where(eq2, col, NUM_EXPERTS), axis=-1, keepdims=True)
    mask2 = col == i2

    g1 = jax.nn.sigmoid(m1 - m2)
    g2 = 1.0 - g1
    gates_ref[...] = jnp.where(mask1, g1, 0.0) + jnp.where(mask2, g2, 0.0)

    p = jnp.exp(logits - m1)
    probs = p / jnp.sum(p, axis=-1, keepdims=True)
    density = jnp.sum(probs, axis=0) / T
    frac = jnp.sum(mask1.astype(jnp.float32) + mask2.astype(jnp.float32),
                   axis=0) / (T * TOP_K)
    loss_ref[...] = (NUM_EXPERTS * jnp.sum(density * frac)).reshape(1, 1)


def _act_quant(v):
    scale = 127.0 / jnp.maximum(jnp.max(jnp.abs(v), axis=-1, keepdims=True), 1e-05)
    return jnp.clip(jnp.round(v * scale), -128.0, 127.0) / scale


def _expert_kernel(x_ref, gates_ref, w_in_ref, w_out_ref, bias_ref, out_ref):
    e = pl.program_id(0)

    @pl.when(e == 0)
    def _init():
        out_ref[...] = jnp.broadcast_to(bias_ref[...], out_ref.shape)

    w_in = w_in_ref[0]
    s_in = 1.0 / jnp.maximum(jnp.mean(jnp.abs(w_in)), 1e-05)
    wq_in = jnp.clip(jnp.round(w_in * s_in), -1.0, 1.0) / s_in
    w_out = w_out_ref[0]
    s_out = 1.0 / jnp.maximum(jnp.mean(jnp.abs(w_out)), 1e-05)
    u_out = jnp.clip(jnp.round(w_out * s_out), -1.0, 1.0).astype(jnp.bfloat16)

    T = x_ref.shape[0]
    col = None
    for t0 in range(0, T, T_CHUNK):
        xs = x_ref[t0:t0 + T_CHUNK, :]
        xq = _act_quant(xs)
        h = jax.lax.dot_general(xq, wq_in, (((1,), (1,)), ((), ())),
                                preferred_element_type=jnp.float32)
        h1 = h[:, :HIDDEN_SIZE]
        g = h[:, HIDDEN_SIZE:]
        glu = h1 * jax.nn.sigmoid(h1) * g
        hsc = 127.0 / jnp.maximum(jnp.max(jnp.abs(glu), axis=-1, keepdims=True), 1e-05)
        qh = jnp.clip(jnp.round(glu * hsc), -128.0, 127.0).astype(jnp.bfloat16)
        o = jax.lax.dot_general(qh, u_out, (((1,), (1,)), ((), ())),
                                preferred_element_type=jnp.float32)
        o = o * (1.0 / (hsc * s_out))
        gs = gates_ref[t0:t0 + T_CHUNK, :]
        if col is None:
            col = jax.lax.broadcasted_iota(jnp.int32, gs.shape, 1)
        g_e = jnp.sum(jnp.where(col == e, gs, 0.0), axis=-1, keepdims=True)
        out_ref[t0:t0 + T_CHUNK, :] += g_e * o


def kernel(x, w_gate, w_in, w_out, bias):
    bsz, length, d = x.shape
    xf = x.reshape(-1, d)
    T = xf.shape[0]

    gates, loss = pl.pallas_call(
        _router_kernel,
        out_shape=(
            jax.ShapeDtypeStruct((T, NUM_EXPERTS), jnp.float32),
            jax.ShapeDtypeStruct((1, 1), jnp.float32),
        ),
    )(xf, w_gate)

    y = pl.pallas_call(
        _expert_kernel,
        grid=(NUM_EXPERTS,),
        in_specs=[
            pl.BlockSpec((T, d), lambda e: (0, 0)),
            pl.BlockSpec((T, NUM_EXPERTS), lambda e: (0, 0)),
            pl.BlockSpec((1, 2 * HIDDEN_SIZE, d), lambda e: (e, 0, 0)),
            pl.BlockSpec((1, d, HIDDEN_SIZE), lambda e: (e, 0, 0)),
            pl.BlockSpec((1, d), lambda e: (0, 0)),
        ],
        out_specs=pl.BlockSpec((T, d), lambda e: (0, 0)),
        out_shape=jax.ShapeDtypeStruct((T, d), jnp.float32),
        compiler_params=pltpu.CompilerParams(
            dimension_semantics=("arbitrary",),
        ),
    )(xf, gates, w_in, w_out, bias.reshape(1, d))

    return y.reshape(bsz, length, d), loss.reshape(())


# speedup vs baseline: 1.0260x; 1.0260x over previous
import functools

import jax
import jax.numpy as jnp
from jax.experimental import pallas as pl
from jax.experimental.pallas import tpu as pltpu

NUM_EXPERTS = 8
TOP_K = 2
INPUT_SIZE = 768
HIDDEN_SIZE = 512
T_CHUNK = 512


def _router_kernel(x_ref, wg_ref, gates_ref, loss_ref):
    x = x_ref[...]
    logits = jnp.dot(x, wg_ref[...], preferred_element_type=jnp.float32)
    T = logits.shape[0]
    col = jax.lax.broadcasted_iota(jnp.int32, logits.shape, 1)

    m1 = jnp.max(logits, axis=-1, keepdims=True)
    eq1 = logits == m1
    i1 = jnp.min(jnp.where(eq1, col, NUM_EXPERTS), axis=-1, keepdims=True)
    mask1 = col == i1

    neg = jnp.full_like(logits, -jnp.inf)
    masked = jnp.where(mask1, neg, logits)
    m2 = jnp.max(masked, axis=-1, keepdims=True)
    eq2 = masked == m2
    i2 = jnp.min(jnp.where(eq2, col, NUM_EXPERTS), axis=-1, keepdims=True)
    mask2 = col == i2

    g1 = jax.nn.sigmoid(m1 - m2)
    g2 = 1.0 - g1
    gates_ref[...] = jnp.where(mask1, g1, 0.0) + jnp.where(mask2, g2, 0.0)

    p = jnp.exp(logits - m1)
    probs = p / jnp.sum(p, axis=-1, keepdims=True)
    density = jnp.sum(probs, axis=0) / T
    frac = jnp.sum(mask1.astype(jnp.float32) + mask2.astype(jnp.float32),
                   axis=0) / (T * TOP_K)
    loss_ref[...] = (NUM_EXPERTS * jnp.sum(density * frac)).reshape(1, 1)


def _act_quant(v):
    scale = 127.0 / jnp.maximum(jnp.max(jnp.abs(v), axis=-1, keepdims=True), 1e-05)
    return jnp.clip(jnp.round(v * scale), -128.0, 127.0) / scale


def _expert_kernel(x_ref, gates_ref, w_in_ref, w_out_ref, bias_ref, out_ref,
                   xq_ref):
    e = pl.program_id(0)

    @pl.when(e == 0)
    def _init():
        out_ref[...] = jnp.broadcast_to(bias_ref[...], out_ref.shape)
        for t0 in range(0, x_ref.shape[0], T_CHUNK):
            xq_ref[t0:t0 + T_CHUNK, :] = _act_quant(x_ref[t0:t0 + T_CHUNK, :])

    w_in = w_in_ref[0]
    s_in = 1.0 / jnp.maximum(jnp.mean(jnp.abs(w_in)), 1e-05)
    wq_in = jnp.clip(jnp.round(w_in * s_in), -1.0, 1.0) / s_in
    w_out = w_out_ref[0]
    s_out = 1.0 / jnp.maximum(jnp.mean(jnp.abs(w_out)), 1e-05)
    u_out = jnp.clip(jnp.round(w_out * s_out), -1.0, 1.0).astype(jnp.bfloat16)

    T = x_ref.shape[0]
    col = None
    for t0 in range(0, T, T_CHUNK):
        xq = xq_ref[t0:t0 + T_CHUNK, :]
        h = jax.lax.dot_general(xq, wq_in, (((1,), (1,)), ((), ())),
                                preferred_element_type=jnp.float32)
        h1 = h[:, :HIDDEN_SIZE]
        g = h[:, HIDDEN_SIZE:]
        glu = h1 * jax.nn.sigmoid(h1) * g
        hsc = 127.0 / jnp.maximum(jnp.max(jnp.abs(glu), axis=-1, keepdims=True), 1e-05)
        qh = jnp.clip(jnp.round(glu * hsc), -128.0, 127.0).astype(jnp.bfloat16)
        o = jax.lax.dot_general(qh, u_out, (((1,), (1,)), ((), ())),
                                preferred_element_type=jnp.float32)
        gs = gates_ref[t0:t0 + T_CHUNK, :]
        if col is None:
            col = jax.lax.broadcasted_iota(jnp.int32, gs.shape, 1)
        g_e = jnp.sum(jnp.where(col == e, gs, 0.0), axis=-1, keepdims=True)
        coef = g_e / (hsc * s_out)
        out_ref[t0:t0 + T_CHUNK, :] += coef * o


def kernel(x, w_gate, w_in, w_out, bias):
    bsz, length, d = x.shape
    xf = x.reshape(-1, d)
    T = xf.shape[0]

    gates, loss = pl.pallas_call(
        _router_kernel,
        out_shape=(
            jax.ShapeDtypeStruct((T, NUM_EXPERTS), jnp.float32),
            jax.ShapeDtypeStruct((1, 1), jnp.float32),
        ),
    )(xf, w_gate)

    y = pl.pallas_call(
        _expert_kernel,
        grid=(NUM_EXPERTS,),
        in_specs=[
            pl.BlockSpec((T, d), lambda e: (0, 0)),
            pl.BlockSpec((T, NUM_EXPERTS), lambda e: (0, 0)),
            pl.BlockSpec((1, 2 * HIDDEN_SIZE, d), lambda e: (e, 0, 0)),
            pl.BlockSpec((1, d, HIDDEN_SIZE), lambda e: (e, 0, 0)),
            pl.BlockSpec((1, d), lambda e: (0, 0)),
        ],
        out_specs=pl.BlockSpec((T, d), lambda e: (0, 0)),
        out_shape=jax.ShapeDtypeStruct((T, d), jnp.float32),
        scratch_shapes=[pltpu.VMEM((T, d), jnp.float32)],
        compiler_params=pltpu.CompilerParams(
            dimension_semantics=("arbitrary",),
        ),
    )(xf, gates, w_in, w_out, bias.reshape(1, d))

    return y.reshape(bsz, length, d), loss.reshape(())


# speedup vs baseline: 1.0803x; 1.0529x over previous
import jax
import jax.numpy as jnp
from jax.experimental import pallas as pl
from jax.experimental.pallas import tpu as pltpu

NUM_EXPERTS = 8
TOP_K = 2
INPUT_SIZE = 768
HIDDEN_SIZE = 512
T_CHUNK = 512


def _act_quant(v):
    scale = 127.0 / jnp.maximum(jnp.max(jnp.abs(v), axis=-1, keepdims=True), 1e-05)
    return jnp.clip(jnp.round(v * scale), -128.0, 127.0) / scale


def _moe_kernel(x_ref, wg_ref, w_in_ref, w_out_ref, bias_ref,
                out_ref, loss_ref, xq_ref, gates_ref):
    e = pl.program_id(0)
    T = x_ref.shape[0]

    @pl.when(e == 0)
    def _router():
        x = x_ref[...]
        logits = jnp.dot(x, wg_ref[...], preferred_element_type=jnp.float32)
        col = jax.lax.broadcasted_iota(jnp.int32, logits.shape, 1)

        m1 = jnp.max(logits, axis=-1, keepdims=True)
        eq1 = logits == m1
        i1 = jnp.min(jnp.where(eq1, col, NUM_EXPERTS), axis=-1, keepdims=True)
        mask1 = col == i1

        neg = jnp.full_like(logits, -jnp.inf)
        masked = jnp.where(mask1, neg, logits)
        m2 = jnp.max(masked, axis=-1, keepdims=True)
        eq2 = masked == m2
        i2 = jnp.min(jnp.where(eq2, col, NUM_EXPERTS), axis=-1, keepdims=True)
        mask2 = col == i2

        g1 = jax.nn.sigmoid(m1 - m2)
        g2 = 1.0 - g1
        gates_ref[...] = jnp.where(mask1, g1, 0.0) + jnp.where(mask2, g2, 0.0)

        p = jnp.exp(logits - m1)
        probs = p / jnp.sum(p, axis=-1, keepdims=True)
        density = jnp.sum(probs, axis=0) / T
        frac = jnp.sum(mask1.astype(jnp.float32) + mask2.astype(jnp.float32),
                       axis=0) / (T * TOP_K)
        loss_ref[...] = (NUM_EXPERTS * jnp.sum(density * frac)).reshape(1, 1)

        out_ref[...] = jnp.broadcast_to(bias_ref[...], out_ref.shape)
        for t0 in range(0, T, T_CHUNK):
            xq_ref[t0:t0 + T_CHUNK, :] = _act_quant(x_ref[t0:t0 + T_CHUNK, :])

    w_in = w_in_ref[0]
    s_in = 1.0 / jnp.maximum(jnp.mean(jnp.abs(w_in)), 1e-05)
    wq_in = jnp.clip(jnp.round(w_in * s_in), -1.0, 1.0) / s_in
    w_out = w_out_ref[0]
    s_out = 1.0 / jnp.maximum(jnp.mean(jnp.abs(w_out)), 1e-05)
    u_out = jnp.clip(jnp.round(w_out * s_out), -1.0, 1.0).astype(jnp.bfloat16)

    col = None
    for t0 in range(0, T, T_CHUNK):
        xq = xq_ref[t0:t0 + T_CHUNK, :]
        h = jax.lax.dot_general(xq, wq_in, (((1,), (1,)), ((), ())),
                                preferred_element_type=jnp.float32)
        h1 = h[:, :HIDDEN_SIZE]
        g = h[:, HIDDEN_SIZE:]
        glu = h1 * jax.nn.sigmoid(h1) * g
        hsc = 127.0 / jnp.maximum(jnp.max(jnp.abs(glu), axis=-1, keepdims=True), 1e-05)
        qh = jnp.clip(jnp.round(glu * hsc), -128.0, 127.0).astype(jnp.bfloat16)
        o = jax.lax.dot_general(qh, u_out, (((1,), (1,)), ((), ())),
                                preferred_element_type=jnp.float32)
        gs = gates_ref[t0:t0 + T_CHUNK, :]
        if col is None:
            col = jax.lax.broadcasted_iota(jnp.int32, gs.shape, 1)
        g_e = jnp.sum(jnp.where(col == e, gs, 0.0), axis=-1, keepdims=True)
        coef = g_e / (hsc * s_out)
        out_ref[t0:t0 + T_CHUNK, :] += coef * o


def kernel(x, w_gate, w_in, w_out, bias):
    bsz, length, d = x.shape
    xf = x.reshape(-1, d)
    T = xf.shape[0]

    y, loss = pl.pallas_call(
        _moe_kernel,
        grid=(NUM_EXPERTS,),
        in_specs=[
            pl.BlockSpec((T, d), lambda e: (0, 0)),
            pl.BlockSpec((d, NUM_EXPERTS), lambda e: (0, 0)),
            pl.BlockSpec((1, 2 * HIDDEN_SIZE, d), lambda e: (e, 0, 0)),
            pl.BlockSpec((1, d, HIDDEN_SIZE), lambda e: (e, 0, 0)),
            pl.BlockSpec((1, d), lambda e: (0, 0)),
        ],
        out_specs=(
            pl.BlockSpec((T, d), lambda e: (0, 0)),
            pl.BlockSpec((1, 1), lambda e: (0, 0)),
        ),
        out_shape=(
            jax.ShapeDtypeStruct((T, d), jnp.float32),
            jax.ShapeDtypeStruct((1, 1), jnp.float32),
        ),
        scratch_shapes=[
            pltpu.VMEM((T, d), jnp.float32),
            pltpu.VMEM((T, NUM_EXPERTS), jnp.float32),
        ],
        compiler_params=pltpu.CompilerParams(
            dimension_semantics=("arbitrary",),
        ),
    )(xf, w_gate, w_in, w_out, bias.reshape(1, d))

    return y.reshape(bsz, length, d), loss.reshape(())


# speedup vs baseline: 1.1349x; 1.0505x over previous
import jax
import jax.numpy as jnp
from jax.experimental import pallas as pl
from jax.experimental.pallas import tpu as pltpu

NUM_EXPERTS = 8
TOP_K = 2
INPUT_SIZE = 768
HIDDEN_SIZE = 512
T_CHUNK = 1024


def _act_quant(v):
    scale = 127.0 / jnp.maximum(jnp.max(jnp.abs(v), axis=-1, keepdims=True), 1e-05)
    return jnp.clip(jnp.round(v * scale), -128.0, 127.0) / scale


def _moe_kernel(x_ref, wg_ref, w_in_ref, w_out_ref, bias_ref,
                out_ref, loss_ref, xq_ref, gates_ref):
    e = pl.program_id(0)
    T = x_ref.shape[0]

    @pl.when(e == 0)
    def _router():
        x = x_ref[...]
        logits = jnp.dot(x, wg_ref[...], preferred_element_type=jnp.float32)
        col = jax.lax.broadcasted_iota(jnp.int32, logits.shape, 1)

        m1 = jnp.max(logits, axis=-1, keepdims=True)
        eq1 = logits == m1
        i1 = jnp.min(jnp.where(eq1, col, NUM_EXPERTS), axis=-1, keepdims=True)
        mask1 = col == i1

        neg = jnp.full_like(logits, -jnp.inf)
        masked = jnp.where(mask1, neg, logits)
        m2 = jnp.max(masked, axis=-1, keepdims=True)
        eq2 = masked == m2
        i2 = jnp.min(jnp.where(eq2, col, NUM_EXPERTS), axis=-1, keepdims=True)
        mask2 = col == i2

        g1 = jax.nn.sigmoid(m1 - m2)
        g2 = 1.0 - g1
        gates_ref[...] = jnp.where(mask1, g1, 0.0) + jnp.where(mask2, g2, 0.0)

        p = jnp.exp(logits - m1)
        probs = p / jnp.sum(p, axis=-1, keepdims=True)
        density = jnp.sum(probs, axis=0) / T
        frac = jnp.sum(mask1.astype(jnp.float32) + mask2.astype(jnp.float32),
                       axis=0) / (T * TOP_K)
        loss_ref[...] = (NUM_EXPERTS * jnp.sum(density * frac)).reshape(1, 1)

        out_ref[...] = jnp.broadcast_to(bias_ref[...], out_ref.shape)
        for t0 in range(0, T, T_CHUNK):
            xq_ref[t0:t0 + T_CHUNK, :] = _act_quant(x_ref[t0:t0 + T_CHUNK, :])

    w_in = w_in_ref[0]
    s_in = 1.0 / jnp.maximum(jnp.mean(jnp.abs(w_in)), 1e-05)
    wq_in = jnp.clip(jnp.round(w_in * s_in), -1.0, 1.0) / s_in
    w_out = w_out_ref[0]
    s_out = 1.0 / jnp.maximum(jnp.mean(jnp.abs(w_out)), 1e-05)
    u_out = jnp.clip(jnp.round(w_out * s_out), -1.0, 1.0).astype(jnp.bfloat16)

    col = None
    for t0 in range(0, T, T_CHUNK):
        xq = xq_ref[t0:t0 + T_CHUNK, :]
        h = jax.lax.dot_general(xq, wq_in, (((1,), (1,)), ((), ())),
                                preferred_element_type=jnp.float32)
        h1 = h[:, :HIDDEN_SIZE]
        g = h[:, HIDDEN_SIZE:]
        glu = h1 * jax.nn.sigmoid(h1) * g
        hsc = 127.0 / jnp.maximum(jnp.max(jnp.abs(glu), axis=-1, keepdims=True), 1e-05)
        qh = jnp.clip(jnp.round(glu * hsc), -128.0, 127.0).astype(jnp.bfloat16)
        o = jax.lax.dot_general(qh, u_out, (((1,), (1,)), ((), ())),
                                preferred_element_type=jnp.float32)
        gs = gates_ref[t0:t0 + T_CHUNK, :]
        if col is None:
            col = jax.lax.broadcasted_iota(jnp.int32, gs.shape, 1)
        g_e = jnp.sum(jnp.where(col == e, gs, 0.0), axis=-1, keepdims=True)
        coef = g_e / (hsc * s_out)
        out_ref[t0:t0 + T_CHUNK, :] += coef * o


def kernel(x, w_gate, w_in, w_out, bias):
    bsz, length, d = x.shape
    xf = x.reshape(-1, d)
    T = xf.shape[0]

    y, loss = pl.pallas_call(
        _moe_kernel,
        grid=(NUM_EXPERTS,),
        in_specs=[
            pl.BlockSpec((T, d), lambda e: (0, 0)),
            pl.BlockSpec((d, NUM_EXPERTS), lambda e: (0, 0)),
            pl.BlockSpec((1, 2 * HIDDEN_SIZE, d), lambda e: (e, 0, 0)),
            pl.BlockSpec((1, d, HIDDEN_SIZE), lambda e: (e, 0, 0)),
            pl.BlockSpec((1, d), lambda e: (0, 0)),
        ],
        out_specs=(
            pl.BlockSpec((T, d), lambda e: (0, 0)),
            pl.BlockSpec((1, 1), lambda e: (0, 0)),
        ),
        out_shape=(
            jax.ShapeDtypeStruct((T, d), jnp.float32),
            jax.ShapeDtypeStruct((1, 1), jnp.float32),
        ),
        scratch_shapes=[
            pltpu.VMEM((T, d), jnp.float32),
            pltpu.VMEM((T, NUM_EXPERTS), jnp.float32),
        ],
        compiler_params=pltpu.CompilerParams(
            dimension_semantics=("arbitrary",),
        ),
    )(xf, w_gate, w_in, w_out, bias.reshape(1, d))

    return y.reshape(bsz, length, d), loss.reshape(())
